# baseline (speedup 1.0000x reference)
import functools

import jax
import jax.numpy as jnp
from jax import lax
from jax.experimental import pallas as pl
from jax.experimental.pallas import tpu as pltpu
from jax.experimental.pallas import tpu_sc as plsc

MAX_POS = 8192
EMBED_DIM = 1024
SEQ_LEN = 8192

_NC = 2
_NS = 16
_NW = _NC * _NS
_ROWS_PER_W = SEQ_LEN // _NW
_CHUNK = 64
_N_CHUNKS = _ROWS_PER_W // _CHUNK

_mesh = plsc.VectorSubcoreMesh(core_axis_name="c", subcore_axis_name="s")


@functools.partial(
    pl.kernel,
    mesh=_mesh,
    out_type=jax.ShapeDtypeStruct((SEQ_LEN, EMBED_DIM), jnp.float32),
    scratch_types=[pltpu.VMEM((_CHUNK, EMBED_DIM), jnp.float32)],
)
def _sc_copy(pe_hbm, out_hbm, buf):
    wid = lax.axis_index("s") * _NC + lax.axis_index("c")
    base = wid * _ROWS_PER_W
    for c in range(_N_CHUNKS):
        off = base + c * _CHUNK
        pltpu.sync_copy(pe_hbm.at[pl.ds(off, _CHUNK), :], buf)
        pltpu.sync_copy(buf, out_hbm.at[pl.ds(off, _CHUNK), :])


def kernel(x, pe):
    return _sc_copy(pe)[None]


# speedup vs baseline: 1.0170x; 1.0170x over previous
import functools

import jax
import jax.numpy as jnp
from jax import lax
from jax.experimental import pallas as pl
from jax.experimental.pallas import tpu as pltpu
from jax.experimental.pallas import tpu_sc as plsc

MAX_POS = 8192
EMBED_DIM = 1024
SEQ_LEN = 8192

_NC = 2
_NS = 16
_NW = _NC * _NS
_ROWS_PER_W = SEQ_LEN // _NW
_CHUNK = 32
_N_CHUNKS = _ROWS_PER_W // _CHUNK

_mesh = plsc.VectorSubcoreMesh(core_axis_name="c", subcore_axis_name="s")


@functools.partial(
    pl.kernel,
    mesh=_mesh,
    out_type=jax.ShapeDtypeStruct((SEQ_LEN, EMBED_DIM), jnp.float32),
    scratch_types=[
        pltpu.VMEM((2, _CHUNK, EMBED_DIM), jnp.float32),
        pltpu.SemaphoreType.DMA((2,)),
        pltpu.SemaphoreType.DMA((2,)),
    ],
)
def _sc_copy(pe_hbm, out_hbm, buf, in_sems, out_sems):
    wid = lax.axis_index("s") * _NC + lax.axis_index("c")
    base = wid * _ROWS_PER_W

    def in_copy(c, s):
        return pltpu.async_copy(
            pe_hbm.at[pl.ds(base + c * _CHUNK, _CHUNK), :],
            buf.at[s],
            in_sems.at[s],
        )

    def out_copy(c, s):
        return pltpu.async_copy(
            buf.at[s],
            out_hbm.at[pl.ds(base + c * _CHUNK, _CHUNK), :],
            out_sems.at[s],
        )

    h_in = [None, None]
    h_out = [None, None]
    h_in[0] = in_copy(0, 0)
    for c in range(_N_CHUNKS):
        s = c & 1
        if c + 1 < _N_CHUNKS:
            o = (c + 1) & 1
            if h_out[o] is not None:
                h_out[o].wait()
            h_in[o] = in_copy(c + 1, o)
        h_in[s].wait()
        h_out[s] = out_copy(c, s)
    h_out[0].wait()
    h_out[1].wait()


def kernel(x, pe):
    return _sc_copy(pe)[None]


# speedup vs baseline: 2.0917x; 2.0568x over previous
import jax
import jax.numpy as jnp
from jax.experimental import pallas as pl
from jax.experimental.pallas import tpu as pltpu

MAX_POS = 8192
EMBED_DIM = 1024
SEQ_LEN = 8192

_N = 4
_CH = SEQ_LEN // _N


def _dma_kernel(pe_hbm, out_hbm, buf, in_sems, out_sems):
    for i in range(_N):
        pltpu.make_async_copy(
            pe_hbm.at[pl.ds(i * _CH, _CH), :], buf.at[i], in_sems.at[i]
        ).start()
    for i in range(_N):
        pltpu.make_async_copy(
            pe_hbm.at[pl.ds(i * _CH, _CH), :], buf.at[i], in_sems.at[i]
        ).wait()
        pltpu.make_async_copy(
            buf.at[i], out_hbm.at[pl.ds(i * _CH, _CH), :], out_sems.at[i]
        ).start()
    for i in range(_N):
        pltpu.make_async_copy(
            buf.at[i], out_hbm.at[pl.ds(i * _CH, _CH), :], out_sems.at[i]
        ).wait()


def kernel(x, pe):
    out = pl.pallas_call(
        _dma_kernel,
        in_specs=[pl.BlockSpec(memory_space=pl.ANY)],
        out_specs=pl.BlockSpec(memory_space=pl.ANY),
        out_shape=jax.ShapeDtypeStruct((SEQ_LEN, EMBED_DIM), pe.dtype),
        scratch_shapes=[
            pltpu.VMEM((_N, _CH, EMBED_DIM), jnp.float32),
            pltpu.SemaphoreType.DMA((_N,)),
            pltpu.SemaphoreType.DMA((_N,)),
        ],
    )(pe)
    return out[None]
